# Initial kernel scaffold; baseline (speedup 1.0000x reference)
import jax
import jax.numpy as jnp
from jax.experimental import pallas as pl


def kernel(inputs, lengths, mask):
    raise NotImplementedError("write your pallas kernel here")


# speedup vs baseline: 4.3160x; 4.3160x over previous
import functools

import jax
import jax.numpy as jnp
from jax import lax
from jax.experimental import pallas as pl
from jax.experimental.pallas import tpu as pltpu
from jax.experimental.pallas import tpu_sc as plsc

B, T, H = 16, 2048, 1024
HH = H // 2
R = 64
NCHUNK = T // R
L = 16
NVEC = HH // L
NG = 2
GV = NVEC // NG

_mesh = plsc.VectorSubcoreMesh(core_axis_name="c", subcore_axis_name="s")


@functools.partial(
    pl.kernel,
    out_type=jax.ShapeDtypeStruct((B, 2 * H), jnp.float32),
    mesh=_mesh,
    compiler_params=pltpu.CompilerParams(needs_layout_passes=False),
    scratch_types=[
        pltpu.VMEM((R, HH), jnp.float32),
        pltpu.VMEM((R,), jnp.float32),
        pltpu.VMEM((2, HH), jnp.float32),
        pltpu.VMEM((L,), jnp.int32),
        pltpu.VMEM((HH,), jnp.float32),
    ],
)
def _pool_sc(x_hbm, wl_hbm, len_hbm, out_hbm, xbuf, wlbuf, accbuf, lenbuf,
             outstage):
    b = lax.axis_index("s")
    hh = lax.axis_index("c")
    c0 = hh * HH

    pltpu.sync_copy(len_hbm, lenbuf)
    lenvi = plsc.load_gather(lenbuf, [jnp.full((L,), b, jnp.int32)])

    zeros = jnp.zeros((L,), jnp.float32)
    ones = jnp.ones((L,), jnp.float32)
    for v in range(2):
        for j in range(NVEC):
            accbuf[v, pl.ds(j * L, L)] = zeros

    def chunk_body(ci, cnt):
        t0 = ci * R
        pltpu.sync_copy(x_hbm.at[b, pl.ds(t0, R), pl.ds(c0, HH)], xbuf)
        pltpu.sync_copy(wl_hbm.at[b, pl.ds(t0, R)], wlbuf)

        for g in range(NG):
            g0 = g * GV * L
            accs = tuple(accbuf[v, pl.ds(g0 + j * L, L)]
                         for v in range(2) for j in range(GV))
            if g == 0:
                accs = accs + (cnt,)

            def row_body(r, carry):
                wlv = plsc.load_gather(wlbuf, [jnp.full((L,), r, jnp.int32)])
                rgv = jnp.full((L,), t0 + r, jnp.int32)
                wgv = jnp.where(rgv < lenvi, ones, zeros)
                out = []
                for j in range(GV):
                    xv = xbuf[r, pl.ds(g0 + j * L, L)]
                    out.append(carry[j] + wgv * xv)
                for j in range(GV):
                    xv = xbuf[r, pl.ds(g0 + j * L, L)]
                    out.append(carry[GV + j] + wlv * xv)
                if g == 0:
                    out.append(carry[2 * GV] + wlv)
                return tuple(out)

            accs = lax.fori_loop(0, R, row_body, accs)
            for v in range(2):
                for j in range(GV):
                    accbuf[v, pl.ds(g0 + j * L, L)] = accs[v * GV + j]
            if g == 0:
                cnt = accs[2 * GV]
        return cnt

    cnt = lax.fori_loop(0, NCHUNK, chunk_body, zeros)

    deng = jnp.maximum(lenvi.astype(jnp.float32), ones)
    denl = jnp.maximum(cnt, ones)
    for v, den in ((0, deng), (1, denl)):
        for j in range(NVEC):
            outstage[pl.ds(j * L, L)] = accbuf[v, pl.ds(j * L, L)] / den
        pltpu.sync_copy(outstage, out_hbm.at[b, pl.ds(v * H + c0, HH)])


def kernel(inputs, lengths, mask):
    wl = mask.astype(jnp.float32)
    lens = lengths.astype(jnp.int32)
    return _pool_sc(inputs, wl, lens)


# speedup vs baseline: 5.3573x; 1.2413x over previous
import functools

import jax
import jax.numpy as jnp
from jax import lax
from jax.experimental import pallas as pl
from jax.experimental.pallas import tpu as pltpu
from jax.experimental.pallas import tpu_sc as plsc

B, T, H = 16, 2048, 1024
HH = H // 2
R = 64
NCHUNK = T // R
L = 16
NVEC = HH // L
NG = 2
GV = NVEC // NG
UNROLL = 2

_mesh = plsc.VectorSubcoreMesh(core_axis_name="c", subcore_axis_name="s")


@functools.partial(
    pl.kernel,
    out_type=jax.ShapeDtypeStruct((B, 2 * H), jnp.float32),
    mesh=_mesh,
    compiler_params=pltpu.CompilerParams(needs_layout_passes=False),
    scratch_types=[
        pltpu.VMEM((R, HH), jnp.float32),
        pltpu.VMEM((R, HH), jnp.float32),
        pltpu.VMEM((R,), jnp.float32),
        pltpu.VMEM((R,), jnp.float32),
        pltpu.VMEM((2, HH), jnp.float32),
        pltpu.VMEM((L,), jnp.int32),
        pltpu.VMEM((HH,), jnp.float32),
        pltpu.SemaphoreType.DMA,
        pltpu.SemaphoreType.DMA,
    ],
)
def _pool_sc(x_hbm, wl_hbm, len_hbm, out_hbm, xbuf0, xbuf1, wlbuf0, wlbuf1,
             accbuf, lenbuf, outstage, sem0, sem1):
    b = lax.axis_index("s")
    hh = lax.axis_index("c")
    c0 = hh * HH

    bufs = ((xbuf0, wlbuf0, sem0), (xbuf1, wlbuf1, sem1))

    def x_copy(ci, xb, sem):
        return pltpu.make_async_copy(
            x_hbm.at[b, pl.ds(ci * R, R), pl.ds(c0, HH)], xb, sem)

    def wl_copy(ci, wlb, sem):
        return pltpu.make_async_copy(wl_hbm.at[b, pl.ds(ci * R, R)], wlb, sem)

    pltpu.sync_copy(len_hbm, lenbuf)
    lenvi = plsc.load_gather(lenbuf, [jnp.full((L,), b, jnp.int32)])

    zeros = jnp.zeros((L,), jnp.float32)
    ones = jnp.ones((L,), jnp.float32)
    for v in range(2):
        for j in range(NVEC):
            accbuf[v, pl.ds(j * L, L)] = zeros

    for par in range(2):
        xb, wlb, sem = bufs[par]
        x_copy(par, xb, sem).start()
        wl_copy(par, wlb, sem).start()

    def chunk2_body(k, cnt):
        for par in range(2):
            ci = 2 * k + par
            xb, wlb, sem = bufs[par]
            t0 = ci * R
            x_copy(ci, xb, sem).wait()
            wl_copy(ci, wlb, sem).wait()

            for q in range(R // L):
                cnt = cnt + wlb[pl.ds(q * L, L)]

            for g in range(NG):
                g0 = g * GV * L
                accs = tuple(accbuf[v, pl.ds(g0 + j * L, L)]
                             for v in range(2) for j in range(GV))

                def row_body(r2, carry):
                    out = list(carry)
                    for dr in range(UNROLL):
                        r = r2 * UNROLL + dr
                        wlv = plsc.load_gather(
                            wlb, [jnp.full((L,), r, jnp.int32)])
                        rgv = jnp.full((L,), t0 + r, jnp.int32)
                        wgv = jnp.where(rgv < lenvi, ones, zeros)
                        for j in range(GV):
                            xv = xb[r, pl.ds(g0 + j * L, L)]
                            out[j] = out[j] + wgv * xv
                            out[GV + j] = out[GV + j] + wlv * xv
                    return tuple(out)

                accs = lax.fori_loop(0, R // UNROLL, row_body, accs)
                for v in range(2):
                    for j in range(GV):
                        accbuf[v, pl.ds(g0 + j * L, L)] = accs[v * GV + j]

            @pl.when(ci + 2 < NCHUNK)
            def _():
                x_copy(ci + 2, xb, sem).start()
                wl_copy(ci + 2, wlb, sem).start()
        return cnt

    cnt = lax.fori_loop(0, NCHUNK // 2, chunk2_body, zeros)

    deng = jnp.maximum(lenvi.astype(jnp.float32), ones)
    denl = jnp.maximum(jnp.full((L,), jnp.sum(cnt)), ones)
    for v, den in ((0, deng), (1, denl)):
        for j in range(NVEC):
            outstage[pl.ds(j * L, L)] = accbuf[v, pl.ds(j * L, L)] / den
        pltpu.sync_copy(outstage, out_hbm.at[b, pl.ds(v * H + c0, HH)])


def kernel(inputs, lengths, mask):
    wl = mask.astype(jnp.float32)
    lens = lengths.astype(jnp.int32)
    return _pool_sc(inputs, wl, lens)


# speedup vs baseline: 6.0071x; 1.1213x over previous
import functools

import jax
import jax.numpy as jnp
from jax import lax
from jax.experimental import pallas as pl
from jax.experimental.pallas import tpu as pltpu
from jax.experimental.pallas import tpu_sc as plsc

B, T, H = 16, 2048, 1024
HH = H // 2
R = 64
NCHUNK = T // R
L = 16
NVEC = HH // L
NG = 2
GV = NVEC // NG
UNROLL = 4

_mesh = plsc.VectorSubcoreMesh(core_axis_name="c", subcore_axis_name="s")


@functools.partial(
    pl.kernel,
    out_type=jax.ShapeDtypeStruct((B, 2 * H), jnp.float32),
    mesh=_mesh,
    compiler_params=pltpu.CompilerParams(needs_layout_passes=False),
    scratch_types=[
        pltpu.VMEM((R, HH), jnp.float32),
        pltpu.VMEM((R, HH), jnp.float32),
        pltpu.VMEM((R,), jnp.float32),
        pltpu.VMEM((R,), jnp.float32),
        pltpu.VMEM((2, HH), jnp.float32),
        pltpu.VMEM((L,), jnp.int32),
        pltpu.VMEM((HH,), jnp.float32),
        pltpu.SemaphoreType.DMA,
        pltpu.SemaphoreType.DMA,
    ],
)
def _pool_sc(x_hbm, wl_hbm, len_hbm, out_hbm, xbuf0, xbuf1, wlbuf0, wlbuf1,
             accbuf, lenbuf, outstage, sem0, sem1):
    b = lax.axis_index("s")
    hh = lax.axis_index("c")
    c0 = hh * HH

    bufs = ((xbuf0, wlbuf0, sem0), (xbuf1, wlbuf1, sem1))

    def x_copy(ci, xb, sem):
        return pltpu.make_async_copy(
            x_hbm.at[b, pl.ds(ci * R, R), pl.ds(c0, HH)], xb, sem)

    def wl_copy(ci, wlb, sem):
        return pltpu.make_async_copy(wl_hbm.at[b, pl.ds(ci * R, R)], wlb, sem)

    pltpu.sync_copy(len_hbm, lenbuf)
    lenvi = plsc.load_gather(lenbuf, [jnp.full((L,), b, jnp.int32)])
    len_s = jnp.max(lenvi)

    zeros = jnp.zeros((L,), jnp.float32)
    ones = jnp.ones((L,), jnp.float32)
    for v in range(2):
        for j in range(NVEC):
            accbuf[v, pl.ds(j * L, L)] = zeros

    for par in range(2):
        xb, wlb, sem = bufs[par]
        x_copy(par, xb, sem).start()
        wl_copy(par, wlb, sem).start()

    def chunk2_body(k, cnt):
        for par in range(2):
            ci = 2 * k + par
            xb, wlb, sem = bufs[par]
            t0 = ci * R
            x_copy(ci, xb, sem).wait()
            wl_copy(ci, wlb, sem).wait()

            for q in range(R // L):
                cnt = cnt + wlb[pl.ds(q * L, L)]

            for g in range(NG):
                g0 = g * GV * L

                @pl.when(t0 >= len_s)
                def _():
                    accs = tuple(accbuf[1, pl.ds(g0 + j * L, L)]
                                 for j in range(GV))

                    def row_body(r2, carry):
                        out = list(carry)
                        for dr in range(UNROLL):
                            r = r2 * UNROLL + dr
                            wlv = plsc.load_gather(
                                wlb, [jnp.full((L,), r, jnp.int32)])
                            for j in range(GV):
                                xv = xb[r, pl.ds(g0 + j * L, L)]
                                out[j] = out[j] + wlv * xv
                        return tuple(out)

                    accs = lax.fori_loop(0, R // UNROLL, row_body, accs)
                    for j in range(GV):
                        accbuf[1, pl.ds(g0 + j * L, L)] = accs[j]

                @pl.when(t0 + R <= len_s)
                def _():
                    accs = tuple(accbuf[v, pl.ds(g0 + j * L, L)]
                                 for v in range(2) for j in range(GV))

                    def row_body(r2, carry):
                        out = list(carry)
                        for dr in range(UNROLL):
                            r = r2 * UNROLL + dr
                            wlv = plsc.load_gather(
                                wlb, [jnp.full((L,), r, jnp.int32)])
                            for j in range(GV):
                                xv = xb[r, pl.ds(g0 + j * L, L)]
                                out[j] = out[j] + xv
                                out[GV + j] = out[GV + j] + wlv * xv
                        return tuple(out)

                    accs = lax.fori_loop(0, R // UNROLL, row_body, accs)
                    for v in range(2):
                        for j in range(GV):
                            accbuf[v, pl.ds(g0 + j * L, L)] = accs[v * GV + j]

                @pl.when(jnp.logical_and(t0 < len_s, t0 + R > len_s))
                def _():
                    accs = tuple(accbuf[v, pl.ds(g0 + j * L, L)]
                                 for v in range(2) for j in range(GV))

                    def row_body(r2, carry):
                        out = list(carry)
                        for dr in range(UNROLL):
                            r = r2 * UNROLL + dr
                            wlv = plsc.load_gather(
                                wlb, [jnp.full((L,), r, jnp.int32)])
                            rgv = jnp.full((L,), t0 + r, jnp.int32)
                            wgv = jnp.where(rgv < lenvi, ones, zeros)
                            for j in range(GV):
                                xv = xb[r, pl.ds(g0 + j * L, L)]
                                out[j] = out[j] + wgv * xv
                                out[GV + j] = out[GV + j] + wlv * xv
                        return tuple(out)

                    accs = lax.fori_loop(0, R // UNROLL, row_body, accs)
                    for v in range(2):
                        for j in range(GV):
                            accbuf[v, pl.ds(g0 + j * L, L)] = accs[v * GV + j]

            @pl.when(ci + 2 < NCHUNK)
            def _():
                x_copy(ci + 2, xb, sem).start()
                wl_copy(ci + 2, wlb, sem).start()
        return cnt

    cnt = lax.fori_loop(0, NCHUNK // 2, chunk2_body, zeros)

    deng = jnp.maximum(lenvi.astype(jnp.float32), ones)
    denl = jnp.maximum(jnp.full((L,), jnp.sum(cnt)), ones)
    for v, den in ((0, deng), (1, denl)):
        for j in range(NVEC):
            outstage[pl.ds(j * L, L)] = accbuf[v, pl.ds(j * L, L)] / den
        pltpu.sync_copy(outstage, out_hbm.at[b, pl.ds(v * H + c0, HH)])


def kernel(inputs, lengths, mask):
    wl = mask.astype(jnp.float32)
    lens = lengths.astype(jnp.int32)
    return _pool_sc(inputs, wl, lens)


# speedup vs baseline: 7.5508x; 1.2570x over previous
import functools

import jax
import jax.numpy as jnp
from jax import lax
from jax.experimental import pallas as pl
from jax.experimental.pallas import tpu as pltpu
from jax.experimental.pallas import tpu_sc as plsc

B, T, H = 16, 2048, 1024
B_SC = 8
B_TC = B - B_SC
NQ = 4
HH = H // NQ
R = 64
NCHUNK = T // R
L = 16
GV = HH // L
UNROLL = 4

_mesh = plsc.VectorSubcoreMesh(core_axis_name="c", subcore_axis_name="s")


@functools.partial(
    pl.kernel,
    out_type=jax.ShapeDtypeStruct((B_SC, 2 * H), jnp.float32),
    mesh=_mesh,
    compiler_params=pltpu.CompilerParams(needs_layout_passes=False),
    scratch_types=[
        pltpu.VMEM((R, HH), jnp.float32),
        pltpu.VMEM((R, HH), jnp.float32),
        pltpu.VMEM((R,), jnp.float32),
        pltpu.VMEM((R,), jnp.float32),
        pltpu.VMEM((2, HH), jnp.float32),
        pltpu.VMEM((L,), jnp.int32),
        pltpu.VMEM((HH,), jnp.float32),
        pltpu.SemaphoreType.DMA,
        pltpu.SemaphoreType.DMA,
    ],
)
def _pool_sc(x_hbm, wl_hbm, len_hbm, out_hbm, xbuf0, xbuf1, wlbuf0, wlbuf1,
             accbuf, lenbuf, outstage, sem0, sem1):
    wid = lax.axis_index("s") * 2 + lax.axis_index("c")
    b = wid // NQ
    c0 = (wid % NQ) * HH

    bufs = ((xbuf0, wlbuf0, sem0), (xbuf1, wlbuf1, sem1))

    def x_copy(ci, xb, sem):
        return pltpu.make_async_copy(
            x_hbm.at[b, pl.ds(ci * R, R), pl.ds(c0, HH)], xb, sem)

    def wl_copy(ci, wlb, sem):
        return pltpu.make_async_copy(wl_hbm.at[b, pl.ds(ci * R, R)], wlb, sem)

    pltpu.sync_copy(len_hbm, lenbuf)
    lenvi = plsc.load_gather(lenbuf, [jnp.full((L,), b, jnp.int32)])
    len_s = jnp.max(lenvi)

    zeros = jnp.zeros((L,), jnp.float32)
    ones = jnp.ones((L,), jnp.float32)
    for v in range(2):
        for j in range(GV):
            accbuf[v, pl.ds(j * L, L)] = zeros

    for par in range(2):
        xb, wlb, sem = bufs[par]
        x_copy(par, xb, sem).start()
        wl_copy(par, wlb, sem).start()

    def chunk2_body(k, cnt):
        for par in range(2):
            ci = 2 * k + par
            xb, wlb, sem = bufs[par]
            t0 = ci * R
            x_copy(ci, xb, sem).wait()
            wl_copy(ci, wlb, sem).wait()

            for q in range(R // L):
                cnt = cnt + wlb[pl.ds(q * L, L)]

            @pl.when(t0 >= len_s)
            def _():
                accs = tuple(accbuf[1, pl.ds(j * L, L)] for j in range(GV))

                def row_body(r2, carry):
                    out = list(carry)
                    for dr in range(UNROLL):
                        r = r2 * UNROLL + dr
                        wlv = plsc.load_gather(
                            wlb, [jnp.full((L,), r, jnp.int32)])
                        for j in range(GV):
                            xv = xb[r, pl.ds(j * L, L)]
                            out[j] = out[j] + wlv * xv
                    return tuple(out)

                accs = lax.fori_loop(0, R // UNROLL, row_body, accs)
                for j in range(GV):
                    accbuf[1, pl.ds(j * L, L)] = accs[j]

            @pl.when(t0 + R <= len_s)
            def _():
                accs = tuple(accbuf[v, pl.ds(j * L, L)]
                             for v in range(2) for j in range(GV))

                def row_body(r2, carry):
                    out = list(carry)
                    for dr in range(UNROLL):
                        r = r2 * UNROLL + dr
                        wlv = plsc.load_gather(
                            wlb, [jnp.full((L,), r, jnp.int32)])
                        for j in range(GV):
                            xv = xb[r, pl.ds(j * L, L)]
                            out[j] = out[j] + xv
                            out[GV + j] = out[GV + j] + wlv * xv
                    return tuple(out)

                accs = lax.fori_loop(0, R // UNROLL, row_body, accs)
                for v in range(2):
                    for j in range(GV):
                        accbuf[v, pl.ds(j * L, L)] = accs[v * GV + j]

            @pl.when(jnp.logical_and(t0 < len_s, t0 + R > len_s))
            def _():
                accs = tuple(accbuf[v, pl.ds(j * L, L)]
                             for v in range(2) for j in range(GV))

                def row_body(r2, carry):
                    out = list(carry)
                    for dr in range(UNROLL):
                        r = r2 * UNROLL + dr
                        wlv = plsc.load_gather(
                            wlb, [jnp.full((L,), r, jnp.int32)])
                        rgv = jnp.full((L,), t0 + r, jnp.int32)
                        wgv = jnp.where(rgv < lenvi, ones, zeros)
                        for j in range(GV):
                            xv = xb[r, pl.ds(j * L, L)]
                            out[j] = out[j] + wgv * xv
                            out[GV + j] = out[GV + j] + wlv * xv
                    return tuple(out)

                accs = lax.fori_loop(0, R // UNROLL, row_body, accs)
                for v in range(2):
                    for j in range(GV):
                        accbuf[v, pl.ds(j * L, L)] = accs[v * GV + j]

            @pl.when(ci + 2 < NCHUNK)
            def _():
                x_copy(ci + 2, xb, sem).start()
                wl_copy(ci + 2, wlb, sem).start()
        return cnt

    cnt = lax.fori_loop(0, NCHUNK // 2, chunk2_body, zeros)

    deng = jnp.maximum(lenvi.astype(jnp.float32), ones)
    denl = jnp.maximum(jnp.full((L,), jnp.sum(cnt)), ones)
    for v, den in ((0, deng), (1, denl)):
        for j in range(GV):
            outstage[pl.ds(j * L, L)] = accbuf[v, pl.ds(j * L, L)] / den
        pltpu.sync_copy(outstage.at[pl.ds(0, HH)],
                        out_hbm.at[b, pl.ds(v * H + c0, HH)])


TBLK = 256
NT = T // TBLK


def _pool_tc_body(x_ref, wg_ref, wl_ref, out_ref, acc_ref, den_ref):
    t = pl.program_id(1)

    @pl.when(t == 0)
    def _():
        acc_ref[...] = jnp.zeros_like(acc_ref)
        den_ref[0] = 0.0
        den_ref[1] = 0.0

    x = x_ref[0]
    wg = wg_ref[0, 0, 0]
    wl = wl_ref[0, 0, 0]
    w2 = jnp.stack([wg, wl], axis=0)
    acc_ref[...] += jnp.dot(w2, x, preferred_element_type=jnp.float32,
                            precision=jax.lax.Precision.HIGHEST)
    den_ref[0] += jnp.sum(wg)
    den_ref[1] += jnp.sum(wl)

    @pl.when(t == NT - 1)
    def _():
        deng = jnp.maximum(den_ref[0], 1.0)
        denl = jnp.maximum(den_ref[1], 1.0)
        out_ref[...] = jnp.concatenate(
            [acc_ref[0:1] / deng, acc_ref[1:2] / denl], axis=1
        ).reshape(1, 1, 2 * H)


_pool_tc = pl.pallas_call(
    _pool_tc_body,
    grid=(B_TC, NT),
    in_specs=[
        pl.BlockSpec((1, TBLK, H), lambda bb, t: (bb + B_SC, t, 0)),
        pl.BlockSpec((1, 1, 1, TBLK), lambda bb, t: (bb + B_SC, t, 0, 0)),
        pl.BlockSpec((1, 1, 1, TBLK), lambda bb, t: (bb + B_SC, t, 0, 0)),
    ],
    out_specs=pl.BlockSpec((1, 1, 2 * H), lambda bb, t: (bb, 0, 0)),
    out_shape=jax.ShapeDtypeStruct((B_TC, 1, 2 * H), jnp.float32),
    scratch_shapes=[
        pltpu.VMEM((2, H), jnp.float32),
        pltpu.SMEM((2,), jnp.float32),
    ],
    compiler_params=pltpu.CompilerParams(
        dimension_semantics=("parallel", "arbitrary")),
)


def kernel(inputs, lengths, mask):
    wl = mask.astype(jnp.float32)
    lens = lengths.astype(jnp.int32)
    wg = (jnp.arange(T, dtype=jnp.int32)[None, :]
          < lens[:, None]).astype(jnp.float32)
    wg4 = wg.reshape(B, NT, 1, TBLK)
    wl4 = wl.reshape(B, NT, 1, TBLK)
    out_sc = _pool_sc(inputs, wl, lens)
    out_tc = _pool_tc(inputs, wg4, wl4).reshape(B_TC, 2 * H)
    return jnp.concatenate([out_sc, out_tc], axis=0)
